# baseline (speedup 1.0000x reference)
import functools

import jax
import jax.numpy as jnp
from jax import lax
from jax.experimental import pallas as pl
from jax.experimental.pallas import tpu as pltpu
from jax.experimental.pallas import tpu_sc as plsc

N = 10000
E = 320000
D = 128
G = 64
NCLS = 10
NC, NS = 2, 16
NW = NC * NS
EB = 128
ET = E + N
S = -(-ET // (NW * EB))
S += S % 2
ECAP = NW * S * EB
DH = D // 2
EB2 = 256
S2 = -(-ET // (NS * EB2))
S2 += S2 % 2
ECAP2 = NS * S2 * EB2
ACC = 12288
RPT = ACC // NS
PB = 3
PCAP = NW * PB * EB
BLK = 1000
C0 = float((1.0 + 1e-5) ** -0.5)


def _deg_body(dsts, degp, didx, onesb, zb, deg_sp):
    c = lax.axis_index("c")
    s = lax.axis_index("s")
    wid = c * NS + s

    def fill(i, carry):
        zb[i, pl.ds(0, 16)] = jnp.zeros((16,), jnp.float32)
        onesb[i, pl.ds(0, 16)] = jnp.ones((16,), jnp.float32)
        return carry

    lax.fori_loop(0, EB, fill, 0)
    for r in range(RPT // EB):
        pltpu.sync_copy(zb, deg_sp.at[pl.ds(s * RPT + r * EB, EB)])
    plsc.subcore_barrier()
    pltpu.sync_copy(dsts.at[wid], didx)

    def body(j, carry):
        pltpu.sync_copy(onesb, deg_sp.at[didx.at[j]], add=True)
        return carry

    lax.fori_loop(0, S, body, 0)
    plsc.subcore_barrier()
    for r in range(RPT // EB):
        pltpu.sync_copy(deg_sp.at[pl.ds(s * RPT + r * EB, EB)],
                        degp.at[c, pl.ds(s * RPT + r * EB, EB)])


def _spmm_body(srcs, dsts, hw, outp, sidx, didx, buf0, buf1, acc_sp, sem0, sem1):
    c = lax.axis_index("c")
    s = lax.axis_index("s")

    def zfill(i, carry):
        for k in range(DH // 16):
            buf0[i, pl.ds(k * 16, 16)] = jnp.zeros((16,), jnp.float32)
        return carry

    lax.fori_loop(0, EB2, zfill, 0)
    for r in range(RPT // EB2):
        pltpu.sync_copy(buf0, acc_sp.at[pl.ds(s * RPT + r * EB2, EB2)])
    plsc.subcore_barrier()
    pltpu.sync_copy(srcs.at[s], sidx)
    pltpu.sync_copy(dsts.at[s], didx)

    pltpu.async_copy(hw.at[c].at[sidx.at[0]], buf0, sem0)
    pltpu.async_copy(hw.at[c].at[sidx.at[1]], buf1, sem1)

    def body(i, carry):
        j = 2 * i
        pltpu.make_async_copy(hw.at[c].at[sidx.at[j]], buf0, sem0).wait()
        pltpu.sync_copy(buf0, acc_sp.at[didx.at[j]], add=True)
        pltpu.async_copy(hw.at[c].at[sidx.at[j + 2]], buf0, sem0)
        pltpu.make_async_copy(hw.at[c].at[sidx.at[j + 1]], buf1, sem1).wait()
        pltpu.sync_copy(buf1, acc_sp.at[didx.at[j + 1]], add=True)
        pltpu.async_copy(hw.at[c].at[sidx.at[j + 3]], buf1, sem1)
        return carry

    lax.fori_loop(0, S2 // 2, body, 0)
    pltpu.make_async_copy(hw.at[c].at[sidx.at[0]], buf0, sem0).wait()
    pltpu.make_async_copy(hw.at[c].at[sidx.at[0]], buf1, sem1).wait()
    plsc.subcore_barrier()
    pltpu.sync_copy(acc_sp.at[pl.ds(s * RPT, RPT)],
                    outp.at[c, pl.ds(s * RPT, RPT)])


def _pool_body(ridx_h, bidx_h, h3, outs, outc,
               ridx, bidx, hbuf, onesb, zb, sums_sp, cnts_sp, sem):
    c = lax.axis_index("c")
    s = lax.axis_index("s")
    wid = c * NS + s

    def zfill(i, carry):
        for k in range(8):
            zb[i, pl.ds(k * 16, 16)] = jnp.zeros((16,), jnp.float32)
        return carry

    lax.fori_loop(0, 8, zfill, 0)

    def ofill(i, carry):
        for k in range(8):
            onesb[i, pl.ds(k * 16, 16)] = jnp.ones((16,), jnp.float32)
        return carry

    lax.fori_loop(0, EB, ofill, 0)
    pltpu.sync_copy(zb, sums_sp.at[pl.ds(s * 8, 8)])
    pltpu.sync_copy(zb, cnts_sp.at[pl.ds(s * 8, 8)])
    plsc.subcore_barrier()
    pltpu.sync_copy(ridx_h.at[wid], ridx)
    pltpu.sync_copy(bidx_h.at[wid], bidx)

    def body(j, carry):
        pltpu.async_copy(h3.at[ridx.at[j]], hbuf, sem).wait()
        pltpu.sync_copy(hbuf, sums_sp.at[bidx.at[j]], add=True)
        pltpu.sync_copy(onesb, cnts_sp.at[bidx.at[j]], add=True)
        return carry

    lax.fori_loop(0, PB, body, 0)
    plsc.subcore_barrier()

    @pl.when(s == 0)
    def _():
        pltpu.sync_copy(sums_sp, outs.at[c])
        pltpu.sync_copy(cnts_sp, outc.at[c])


def _first_body(x_ref, degp_ref, w_ref, b_ref, dinv_ref, hw_ref):
    deg = degp_ref[0, :, 0:1] + degp_ref[1, :, 0:1]
    dinv = jnp.where(deg > 0, lax.rsqrt(jnp.maximum(deg, 1e-30)), 0.0)
    dinvb = jnp.broadcast_to(dinv, (BLK, D))
    dinv_ref[...] = dinvb
    hw = jnp.dot(x_ref[...], w_ref[...], preferred_element_type=jnp.float32)
    hw = (hw + b_ref[...]) * dinvb
    hw_ref[0] = hw[:, :DH]
    hw_ref[1] = hw[:, DH:]


def _mid_body(p_ref, prev_ref, dinv_ref, g_ref, bt_ref, w_ref, b_ref,
              hn_ref, hw_ref):
    dinv = dinv_ref[...]
    sv = jnp.concatenate([p_ref[0], p_ref[1]], axis=-1) * dinv
    hn = jnp.maximum(g_ref[...] * C0 * sv + bt_ref[...], 0.0) + prev_ref[...]
    hn_ref[...] = hn
    hw = jnp.dot(hn, w_ref[...], preferred_element_type=jnp.float32)
    hw = (hw + b_ref[...]) * dinv
    hw_ref[0] = hw[:, :DH]
    hw_ref[1] = hw[:, DH:]


def _last_body(p_ref, prev_ref, dinv_ref, g_ref, bt_ref, hn_ref):
    sv = jnp.concatenate([p_ref[0], p_ref[1]], axis=-1) * dinv_ref[...]
    hn_ref[...] = (jnp.maximum(g_ref[...] * C0 * sv + bt_ref[...], 0.0)
                   + prev_ref[...])


def _head_body(sums_ref, cnts_ref, wc_ref, bc_ref, out_ref):
    sv = sums_ref[0] + sums_ref[1]
    cv = cnts_ref[0] + cnts_ref[1]
    emb = sv / jnp.maximum(cv, 1.0)
    out_ref[...] = (jnp.dot(emb, wc_ref[...], preferred_element_type=jnp.float32)
                    + bc_ref[...])


def _f32(shape):
    return jax.ShapeDtypeStruct(shape, jnp.float32)


@functools.lru_cache(maxsize=None)
def _build():
    mesh = plsc.VectorSubcoreMesh(core_axis_name="c", subcore_axis_name="s")

    deg_k = functools.partial(
        pl.kernel, _deg_body,
        out_type=_f32((NC, ACC, 16)),
        mesh=mesh,
        compiler_params=pltpu.CompilerParams(use_tc_tiling_on_sc=False),
        scratch_types=[
            pltpu.VMEM((S, EB), jnp.int32),
            pltpu.VMEM((EB, 16), jnp.float32),
            pltpu.VMEM((EB, 16), jnp.float32),
            pltpu.VMEM_SHARED((ACC, 16), jnp.float32),
        ],
    )()

    spmm_k = functools.partial(
        pl.kernel, _spmm_body,
        out_type=_f32((NC, ACC, DH)),
        mesh=mesh,
        compiler_params=pltpu.CompilerParams(use_tc_tiling_on_sc=False),
        scratch_types=[
            pltpu.VMEM((S2 + 2, EB2), jnp.int32),
            pltpu.VMEM((S2, EB2), jnp.int32),
            pltpu.VMEM((EB2, DH), jnp.float32),
            pltpu.VMEM((EB2, DH), jnp.float32),
            pltpu.VMEM_SHARED((ACC, DH), jnp.float32),
            pltpu.SemaphoreType.DMA,
            pltpu.SemaphoreType.DMA,
        ],
    )()

    pool_k = functools.partial(
        pl.kernel, _pool_body,
        out_type=(_f32((NC, 128, D)), _f32((NC, 128, D))),
        mesh=mesh,
        compiler_params=pltpu.CompilerParams(use_tc_tiling_on_sc=False),
        scratch_types=[
            pltpu.VMEM((PB, EB), jnp.int32),
            pltpu.VMEM((PB, EB), jnp.int32),
            pltpu.VMEM((EB, D), jnp.float32),
            pltpu.VMEM((EB, D), jnp.float32),
            pltpu.VMEM((8, D), jnp.float32),
            pltpu.VMEM_SHARED((128, D), jnp.float32),
            pltpu.VMEM_SHARED((128, D), jnp.float32),
            pltpu.SemaphoreType.DMA,
        ],
    )()

    grid = (N // BLK,)
    vec_spec = pl.BlockSpec((1, D), lambda j: (0, 0))
    row_spec = pl.BlockSpec((BLK, D), lambda j: (j, 0))
    mat_spec = pl.BlockSpec((D, D), lambda j: (0, 0))
    p_spec = pl.BlockSpec((NC, BLK, DH), lambda j: (0, j, 0))
    hw_spec = pl.BlockSpec((NC, BLK, DH), lambda j: (0, j, 0))
    hw_shape = _f32((NC, N, DH))

    first_k = pl.pallas_call(
        _first_body,
        grid=grid,
        in_specs=[row_spec,
                  pl.BlockSpec((NC, BLK, 16), lambda j: (0, j, 0)),
                  mat_spec, vec_spec],
        out_specs=[row_spec, hw_spec],
        out_shape=[_f32((N, D)), hw_shape],
    )

    mid_k = pl.pallas_call(
        _mid_body,
        grid=grid,
        in_specs=[p_spec, row_spec, row_spec, vec_spec, vec_spec,
                  mat_spec, vec_spec],
        out_specs=[row_spec, hw_spec],
        out_shape=[_f32((N, D)), hw_shape],
    )

    last_k = pl.pallas_call(
        _last_body,
        grid=grid,
        in_specs=[p_spec, row_spec, row_spec, vec_spec, vec_spec],
        out_specs=row_spec,
        out_shape=_f32((N, D)),
    )

    head_k = pl.pallas_call(
        _head_body,
        in_specs=[pl.BlockSpec((NC, 128, D), lambda: (0, 0, 0)),
                  pl.BlockSpec((NC, 128, D), lambda: (0, 0, 0)),
                  pl.BlockSpec((D, D), lambda: (0, 0)),
                  pl.BlockSpec((1, D), lambda: (0, 0))],
        out_specs=pl.BlockSpec((128, D), lambda: (0, 0)),
        out_shape=_f32((128, D)),
    )

    return deg_k, spmm_k, pool_k, first_k, mid_k, last_k, head_k


def kernel(x, edge_index, batch, W1, b1, g1, bt1, W2, b2, g2, bt2,
           W3, b3, g3, bt3, Wc, bc):
    deg_k, spmm_k, pool_k, first_k, mid_k, last_k, head_k = _build()

    loop = jnp.arange(N, dtype=jnp.int32)
    src = jnp.concatenate([edge_index[0], loop])
    dst = jnp.concatenate([edge_index[1], loop])
    dsts = jnp.pad(dst, (0, ECAP - ET),
                   constant_values=N).reshape(NW, S, EB)
    srcs2 = jnp.pad(src, (0, ECAP2 - ET)).reshape(NS, S2, EB2)
    srcs2 = jnp.concatenate(
        [srcs2, jnp.zeros((NS, 2, EB2), jnp.int32)], axis=1)
    dsts2 = jnp.pad(dst, (0, ECAP2 - ET),
                    constant_values=N).reshape(NS, S2, EB2)

    degp = deg_k(dsts)

    b1r = b1.reshape(1, D)
    dinvf, hw = first_k(x, degp, W1, b1r)

    p = spmm_k(srcs2, dsts2, hw)
    h1, hw = mid_k(p, x, dinvf, g1.reshape(1, D), bt1.reshape(1, D),
                   W2, b2.reshape(1, D))
    p = spmm_k(srcs2, dsts2, hw)
    h2, hw = mid_k(p, h1, dinvf, g2.reshape(1, D), bt2.reshape(1, D),
                   W3, b3.reshape(1, D))
    p = spmm_k(srcs2, dsts2, hw)
    h3 = last_k(p, h2, dinvf, g3.reshape(1, D), bt3.reshape(1, D))

    rows = jnp.minimum(jnp.arange(PCAP, dtype=jnp.int32), N - 1)
    rows = rows.reshape(NW, PB, EB)
    bpad = jnp.pad(batch, (0, PCAP - N), constant_values=G).reshape(NW, PB, EB)
    sums, cnts = pool_k(rows, bpad, h3)

    wcp = jnp.pad(Wc, ((0, 0), (0, D - NCLS)))
    bcp = jnp.pad(bc, (0, D - NCLS)).reshape(1, D)
    logits = head_k(sums, cnts, wcp, bcp)
    return logits[:G, :NCLS]


# speedup vs baseline: 1.0315x; 1.0315x over previous
import functools

import jax
import jax.numpy as jnp
from jax import lax
from jax.experimental import pallas as pl
from jax.experimental.pallas import tpu as pltpu
from jax.experimental.pallas import tpu_sc as plsc

N = 10000
E = 320000
D = 128
G = 64
NCLS = 10
NC, NS = 2, 16
NW = NC * NS
EB = 128
ET = E + N
S = -(-ET // (NW * EB))
S += S % 2
ECAP = NW * S * EB
DH = D // 2
EB2 = 128
NB = 4
S2 = -(-ET // (NS * EB2))
S2 += (-S2) % NB
ECAP2 = NS * S2 * EB2
ACC = 10240
RPT = ACC // NS
PB = 3
PCAP = NW * PB * EB
BLK = 1000
C0 = float((1.0 + 1e-5) ** -0.5)


def _deg_body(dsts, degp, didx, onesb, zb, deg_sp):
    c = lax.axis_index("c")
    s = lax.axis_index("s")
    wid = c * NS + s

    def fill(i, carry):
        zb[i, pl.ds(0, 16)] = jnp.zeros((16,), jnp.float32)
        onesb[i, pl.ds(0, 16)] = jnp.ones((16,), jnp.float32)
        return carry

    lax.fori_loop(0, EB, fill, 0)
    for r in range(RPT // EB):
        pltpu.sync_copy(zb, deg_sp.at[pl.ds(s * RPT + r * EB, EB)])
    plsc.subcore_barrier()
    pltpu.sync_copy(dsts.at[wid], didx)

    def body(j, carry):
        pltpu.sync_copy(onesb, deg_sp.at[didx.at[j]], add=True)
        return carry

    lax.fori_loop(0, S, body, 0)
    plsc.subcore_barrier()
    for r in range(RPT // EB):
        pltpu.sync_copy(deg_sp.at[pl.ds(s * RPT + r * EB, EB)],
                        degp.at[c, pl.ds(s * RPT + r * EB, EB)])


def _spmm_body(srcs, dsts, hw, outp, sidx, didx,
               buf0, buf1, buf2, buf3, acc_sp, sem0, sem1, sem2, sem3):
    c = lax.axis_index("c")
    s = lax.axis_index("s")
    bufs = (buf0, buf1, buf2, buf3)
    sems = (sem0, sem1, sem2, sem3)

    def zfill(i, carry):
        for k in range(DH // 16):
            buf0[i, pl.ds(k * 16, 16)] = jnp.zeros((16,), jnp.float32)
        return carry

    lax.fori_loop(0, EB2, zfill, 0)
    for r in range(RPT // EB2):
        pltpu.sync_copy(buf0, acc_sp.at[pl.ds(s * RPT + r * EB2, EB2)])
    plsc.subcore_barrier()
    pltpu.sync_copy(srcs.at[s], sidx)
    pltpu.sync_copy(dsts.at[s], didx)

    for b in range(NB):
        pltpu.async_copy(hw.at[c].at[sidx.at[b]], bufs[b], sems[b])

    def body(i, carry):
        j = NB * i
        for b in range(NB):
            pltpu.make_async_copy(hw.at[c].at[sidx.at[j + b]],
                                  bufs[b], sems[b]).wait()
            pltpu.sync_copy(bufs[b], acc_sp.at[didx.at[j + b]], add=True)
            pltpu.async_copy(hw.at[c].at[sidx.at[j + b + NB]],
                             bufs[b], sems[b])
        return carry

    lax.fori_loop(0, S2 // NB, body, 0)
    for b in range(NB):
        pltpu.make_async_copy(hw.at[c].at[sidx.at[0]], bufs[b], sems[b]).wait()
    plsc.subcore_barrier()
    pltpu.sync_copy(acc_sp.at[pl.ds(s * RPT, RPT)],
                    outp.at[c, pl.ds(s * RPT, RPT)])


def _pool_body(ridx_h, bidx_h, h3, outs, outc,
               ridx, bidx, hbuf, onesb, zb, sums_sp, cnts_sp, sem):
    c = lax.axis_index("c")
    s = lax.axis_index("s")
    wid = c * NS + s

    def zfill(i, carry):
        for k in range(8):
            zb[i, pl.ds(k * 16, 16)] = jnp.zeros((16,), jnp.float32)
        return carry

    lax.fori_loop(0, 8, zfill, 0)

    def ofill(i, carry):
        for k in range(8):
            onesb[i, pl.ds(k * 16, 16)] = jnp.ones((16,), jnp.float32)
        return carry

    lax.fori_loop(0, EB, ofill, 0)
    pltpu.sync_copy(zb, sums_sp.at[pl.ds(s * 8, 8)])
    pltpu.sync_copy(zb, cnts_sp.at[pl.ds(s * 8, 8)])
    plsc.subcore_barrier()
    pltpu.sync_copy(ridx_h.at[wid], ridx)
    pltpu.sync_copy(bidx_h.at[wid], bidx)

    def body(j, carry):
        pltpu.async_copy(h3.at[ridx.at[j]], hbuf, sem).wait()
        pltpu.sync_copy(hbuf, sums_sp.at[bidx.at[j]], add=True)
        pltpu.sync_copy(onesb, cnts_sp.at[bidx.at[j]], add=True)
        return carry

    lax.fori_loop(0, PB, body, 0)
    plsc.subcore_barrier()

    @pl.when(s == 0)
    def _():
        pltpu.sync_copy(sums_sp, outs.at[c])
        pltpu.sync_copy(cnts_sp, outc.at[c])


def _first_body(x_ref, degp_ref, w_ref, b_ref, dinv_ref, hw_ref):
    deg = degp_ref[0, :, 0:1] + degp_ref[1, :, 0:1]
    dinv = jnp.where(deg > 0, lax.rsqrt(jnp.maximum(deg, 1e-30)), 0.0)
    dinvb = jnp.broadcast_to(dinv, (BLK, D))
    dinv_ref[...] = dinvb
    hw = jnp.dot(x_ref[...], w_ref[...], preferred_element_type=jnp.float32)
    hw = (hw + b_ref[...]) * dinvb
    hw_ref[0] = hw[:, :DH]
    hw_ref[1] = hw[:, DH:]


def _mid_body(p_ref, prev_ref, dinv_ref, g_ref, bt_ref, w_ref, b_ref,
              hn_ref, hw_ref):
    dinv = dinv_ref[...]
    sv = jnp.concatenate([p_ref[0], p_ref[1]], axis=-1) * dinv
    hn = jnp.maximum(g_ref[...] * C0 * sv + bt_ref[...], 0.0) + prev_ref[...]
    hn_ref[...] = hn
    hw = jnp.dot(hn, w_ref[...], preferred_element_type=jnp.float32)
    hw = (hw + b_ref[...]) * dinv
    hw_ref[0] = hw[:, :DH]
    hw_ref[1] = hw[:, DH:]


def _last_body(p_ref, prev_ref, dinv_ref, g_ref, bt_ref, hn_ref):
    sv = jnp.concatenate([p_ref[0], p_ref[1]], axis=-1) * dinv_ref[...]
    hn_ref[...] = (jnp.maximum(g_ref[...] * C0 * sv + bt_ref[...], 0.0)
                   + prev_ref[...])


def _head_body(sums_ref, cnts_ref, wc_ref, bc_ref, out_ref):
    sv = sums_ref[0] + sums_ref[1]
    cv = cnts_ref[0] + cnts_ref[1]
    emb = sv / jnp.maximum(cv, 1.0)
    out_ref[...] = (jnp.dot(emb, wc_ref[...], preferred_element_type=jnp.float32)
                    + bc_ref[...])


def _f32(shape):
    return jax.ShapeDtypeStruct(shape, jnp.float32)


@functools.lru_cache(maxsize=None)
def _build():
    mesh = plsc.VectorSubcoreMesh(core_axis_name="c", subcore_axis_name="s")

    deg_k = functools.partial(
        pl.kernel, _deg_body,
        out_type=_f32((NC, ACC, 16)),
        mesh=mesh,
        compiler_params=pltpu.CompilerParams(use_tc_tiling_on_sc=False),
        scratch_types=[
            pltpu.VMEM((S, EB), jnp.int32),
            pltpu.VMEM((EB, 16), jnp.float32),
            pltpu.VMEM((EB, 16), jnp.float32),
            pltpu.VMEM_SHARED((ACC, 16), jnp.float32),
        ],
    )()

    spmm_k = functools.partial(
        pl.kernel, _spmm_body,
        out_type=_f32((NC, ACC, DH)),
        mesh=mesh,
        compiler_params=pltpu.CompilerParams(use_tc_tiling_on_sc=False),
        scratch_types=[
            pltpu.VMEM((S2 + NB, EB2), jnp.int32),
            pltpu.VMEM((S2, EB2), jnp.int32),
            pltpu.VMEM((EB2, DH), jnp.float32),
            pltpu.VMEM((EB2, DH), jnp.float32),
            pltpu.VMEM((EB2, DH), jnp.float32),
            pltpu.VMEM((EB2, DH), jnp.float32),
            pltpu.VMEM_SHARED((ACC, DH), jnp.float32),
            pltpu.SemaphoreType.DMA,
            pltpu.SemaphoreType.DMA,
            pltpu.SemaphoreType.DMA,
            pltpu.SemaphoreType.DMA,
        ],
    )()

    pool_k = functools.partial(
        pl.kernel, _pool_body,
        out_type=(_f32((NC, 128, D)), _f32((NC, 128, D))),
        mesh=mesh,
        compiler_params=pltpu.CompilerParams(use_tc_tiling_on_sc=False),
        scratch_types=[
            pltpu.VMEM((PB, EB), jnp.int32),
            pltpu.VMEM((PB, EB), jnp.int32),
            pltpu.VMEM((EB, D), jnp.float32),
            pltpu.VMEM((EB, D), jnp.float32),
            pltpu.VMEM((8, D), jnp.float32),
            pltpu.VMEM_SHARED((128, D), jnp.float32),
            pltpu.VMEM_SHARED((128, D), jnp.float32),
            pltpu.SemaphoreType.DMA,
        ],
    )()

    grid = (N // BLK,)
    vec_spec = pl.BlockSpec((1, D), lambda j: (0, 0))
    row_spec = pl.BlockSpec((BLK, D), lambda j: (j, 0))
    mat_spec = pl.BlockSpec((D, D), lambda j: (0, 0))
    p_spec = pl.BlockSpec((NC, BLK, DH), lambda j: (0, j, 0))
    hw_spec = pl.BlockSpec((NC, BLK, DH), lambda j: (0, j, 0))
    hw_shape = _f32((NC, N, DH))

    first_k = pl.pallas_call(
        _first_body,
        grid=grid,
        in_specs=[row_spec,
                  pl.BlockSpec((NC, BLK, 16), lambda j: (0, j, 0)),
                  mat_spec, vec_spec],
        out_specs=[row_spec, hw_spec],
        out_shape=[_f32((N, D)), hw_shape],
    )

    mid_k = pl.pallas_call(
        _mid_body,
        grid=grid,
        in_specs=[p_spec, row_spec, row_spec, vec_spec, vec_spec,
                  mat_spec, vec_spec],
        out_specs=[row_spec, hw_spec],
        out_shape=[_f32((N, D)), hw_shape],
    )

    last_k = pl.pallas_call(
        _last_body,
        grid=grid,
        in_specs=[p_spec, row_spec, row_spec, vec_spec, vec_spec],
        out_specs=row_spec,
        out_shape=_f32((N, D)),
    )

    head_k = pl.pallas_call(
        _head_body,
        in_specs=[pl.BlockSpec((NC, 128, D), lambda: (0, 0, 0)),
                  pl.BlockSpec((NC, 128, D), lambda: (0, 0, 0)),
                  pl.BlockSpec((D, D), lambda: (0, 0)),
                  pl.BlockSpec((1, D), lambda: (0, 0))],
        out_specs=pl.BlockSpec((128, D), lambda: (0, 0)),
        out_shape=_f32((128, D)),
    )

    return deg_k, spmm_k, pool_k, first_k, mid_k, last_k, head_k


def kernel(x, edge_index, batch, W1, b1, g1, bt1, W2, b2, g2, bt2,
           W3, b3, g3, bt3, Wc, bc):
    deg_k, spmm_k, pool_k, first_k, mid_k, last_k, head_k = _build()

    loop = jnp.arange(N, dtype=jnp.int32)
    src = jnp.concatenate([edge_index[0], loop])
    dst = jnp.concatenate([edge_index[1], loop])
    dsts = jnp.pad(dst, (0, ECAP - ET),
                   constant_values=N).reshape(NW, S, EB)
    srcs2 = jnp.pad(src, (0, ECAP2 - ET)).reshape(NS, S2, EB2)
    srcs2 = jnp.concatenate(
        [srcs2, jnp.zeros((NS, NB, EB2), jnp.int32)], axis=1)
    dsts2 = jnp.pad(dst, (0, ECAP2 - ET),
                    constant_values=N).reshape(NS, S2, EB2)

    degp = deg_k(dsts)

    b1r = b1.reshape(1, D)
    dinvf, hw = first_k(x, degp, W1, b1r)

    p = spmm_k(srcs2, dsts2, hw)
    h1, hw = mid_k(p, x, dinvf, g1.reshape(1, D), bt1.reshape(1, D),
                   W2, b2.reshape(1, D))
    p = spmm_k(srcs2, dsts2, hw)
    h2, hw = mid_k(p, h1, dinvf, g2.reshape(1, D), bt2.reshape(1, D),
                   W3, b3.reshape(1, D))
    p = spmm_k(srcs2, dsts2, hw)
    h3 = last_k(p, h2, dinvf, g3.reshape(1, D), bt3.reshape(1, D))

    rows = jnp.minimum(jnp.arange(PCAP, dtype=jnp.int32), N - 1)
    rows = rows.reshape(NW, PB, EB)
    bpad = jnp.pad(batch, (0, PCAP - N), constant_values=G).reshape(NW, PB, EB)
    sums, cnts = pool_k(rows, bpad, h3)

    wcp = jnp.pad(Wc, ((0, 0), (0, D - NCLS)))
    bcp = jnp.pad(bc, (0, D - NCLS)).reshape(1, D)
    logits = head_k(sums, cnts, wcp, bcp)
    return logits[:G, :NCLS]


# speedup vs baseline: 1.5202x; 1.4738x over previous
import functools

import jax
import jax.numpy as jnp
from jax import lax
from jax.experimental import pallas as pl
from jax.experimental.pallas import tpu as pltpu
from jax.experimental.pallas import tpu_sc as plsc

N = 10000
E = 320000
D = 128
G = 64
NCLS = 10
NC, NS = 2, 16
NW = NC * NS
EB = 128
ET = E + N
S = -(-ET // (NW * EB))
S += S % 2
ECAP = NW * S * EB
DH = D // 2
EB2 = 128
NB = 2
S2 = -(-ET // (NS * EB2))
S2 += (-S2) % NB
ECAP2 = NS * S2 * EB2
ACC = 10240
RPT = ACC // NS
PB = 3
PCAP = NW * PB * EB
BLK = 1000
C0 = float((1.0 + 1e-5) ** -0.5)


def _deg_body(dsts, degp, didx, onesb, zb, deg_sp):
    c = lax.axis_index("c")
    s = lax.axis_index("s")
    wid = c * NS + s

    def fill(i, carry):
        zb[i, pl.ds(0, 16)] = jnp.zeros((16,), jnp.float32)
        onesb[i, pl.ds(0, 16)] = jnp.ones((16,), jnp.float32)
        return carry

    lax.fori_loop(0, EB, fill, 0)
    for r in range(RPT // EB):
        pltpu.sync_copy(zb, deg_sp.at[pl.ds(s * RPT + r * EB, EB)])
    plsc.subcore_barrier()
    pltpu.sync_copy(dsts.at[wid], didx)

    def body(j, carry):
        pltpu.sync_copy(onesb, deg_sp.at[didx.at[j]], add=True)
        return carry

    lax.fori_loop(0, S, body, 0)
    plsc.subcore_barrier()
    for r in range(RPT // EB):
        pltpu.sync_copy(deg_sp.at[pl.ds(s * RPT + r * EB, EB)],
                        degp.at[c, pl.ds(s * RPT + r * EB, EB)])


def _spmm_body(srcs, dsts, hw, outp, sidx, didx,
               buf0, buf1, acc_sp, sem0, sem1):
    c = lax.axis_index("c")
    s = lax.axis_index("s")
    bufs = (buf0, buf1)
    sems = (sem0, sem1)

    def zfill(i, carry):
        for k in range(DH // 16):
            buf0[i, pl.ds(k * 16, 16)] = jnp.zeros((16,), jnp.float32)
        return carry

    lax.fori_loop(0, EB2, zfill, 0)
    for r in range(RPT // EB2):
        pltpu.sync_copy(buf0, acc_sp.at[pl.ds(s * RPT + r * EB2, EB2)])
    plsc.subcore_barrier()
    pltpu.sync_copy(srcs.at[s], sidx)
    pltpu.sync_copy(dsts.at[s], didx)

    for b in range(NB):
        pltpu.async_copy(hw.at[c].at[sidx.at[b]], bufs[b], sems[b])

    def body(i, carry):
        j = NB * i
        for b in range(NB):
            pltpu.make_async_copy(hw.at[c].at[sidx.at[j + b]],
                                  bufs[b], sems[b]).wait()
            pltpu.sync_copy(bufs[b], acc_sp.at[didx.at[j + b]], add=True)
            pltpu.async_copy(hw.at[c].at[sidx.at[j + b + NB]],
                             bufs[b], sems[b])
        return carry

    lax.fori_loop(0, S2 // NB, body, 0)
    for b in range(NB):
        pltpu.make_async_copy(hw.at[c].at[sidx.at[0]], bufs[b], sems[b]).wait()
    plsc.subcore_barrier()
    pltpu.sync_copy(acc_sp.at[pl.ds(s * RPT, RPT)],
                    outp.at[c, pl.ds(s * RPT, RPT)])


def _pool_body(ridx_h, bidx_h, h3, outs, outc,
               ridx, bidx, hbuf, onesb, zb, sums_sp, cnts_sp, sem):
    c = lax.axis_index("c")
    s = lax.axis_index("s")
    wid = c * NS + s

    def zfill(i, carry):
        for k in range(8):
            zb[i, pl.ds(k * 16, 16)] = jnp.zeros((16,), jnp.float32)
        return carry

    lax.fori_loop(0, 8, zfill, 0)

    def ofill(i, carry):
        for k in range(8):
            onesb[i, pl.ds(k * 16, 16)] = jnp.ones((16,), jnp.float32)
        return carry

    lax.fori_loop(0, EB, ofill, 0)
    pltpu.sync_copy(zb, sums_sp.at[pl.ds(s * 8, 8)])
    pltpu.sync_copy(zb, cnts_sp.at[pl.ds(s * 8, 8)])
    plsc.subcore_barrier()
    pltpu.sync_copy(ridx_h.at[wid], ridx)
    pltpu.sync_copy(bidx_h.at[wid], bidx)

    def body(j, carry):
        pltpu.async_copy(h3.at[ridx.at[j]], hbuf, sem).wait()
        pltpu.sync_copy(hbuf, sums_sp.at[bidx.at[j]], add=True)
        pltpu.sync_copy(onesb, cnts_sp.at[bidx.at[j]], add=True)
        return carry

    lax.fori_loop(0, PB, body, 0)
    plsc.subcore_barrier()

    @pl.when(s == 0)
    def _():
        pltpu.sync_copy(sums_sp, outs.at[c])
        pltpu.sync_copy(cnts_sp, outc.at[c])


def _first_body(x_ref, degp_ref, w_ref, b_ref, dinv_ref, hw_ref):
    deg = degp_ref[0, :, 0:1] + degp_ref[1, :, 0:1]
    dinv = jnp.where(deg > 0, lax.rsqrt(jnp.maximum(deg, 1e-30)), 0.0)
    dinvb = jnp.broadcast_to(dinv, (BLK, D))
    dinv_ref[...] = dinvb
    hw = jnp.dot(x_ref[...], w_ref[...], preferred_element_type=jnp.float32)
    hw = (hw + b_ref[...]) * dinvb
    hw_ref[0] = hw[:, :DH]
    hw_ref[1] = hw[:, DH:]


def _mid_body(p_ref, prev_ref, dinv_ref, g_ref, bt_ref, w_ref, b_ref,
              hn_ref, hw_ref):
    dinv = dinv_ref[...]
    sv = jnp.concatenate([p_ref[0], p_ref[1]], axis=-1) * dinv
    hn = jnp.maximum(g_ref[...] * C0 * sv + bt_ref[...], 0.0) + prev_ref[...]
    hn_ref[...] = hn
    hw = jnp.dot(hn, w_ref[...], preferred_element_type=jnp.float32)
    hw = (hw + b_ref[...]) * dinv
    hw_ref[0] = hw[:, :DH]
    hw_ref[1] = hw[:, DH:]


def _last_body(p_ref, prev_ref, dinv_ref, g_ref, bt_ref, hn_ref):
    sv = jnp.concatenate([p_ref[0], p_ref[1]], axis=-1) * dinv_ref[...]
    hn_ref[...] = (jnp.maximum(g_ref[...] * C0 * sv + bt_ref[...], 0.0)
                   + prev_ref[...])


def _head_body(sums_ref, cnts_ref, wc_ref, bc_ref, out_ref):
    sv = sums_ref[0] + sums_ref[1]
    cv = cnts_ref[0] + cnts_ref[1]
    emb = sv / jnp.maximum(cv, 1.0)
    out_ref[...] = (jnp.dot(emb, wc_ref[...], preferred_element_type=jnp.float32)
                    + bc_ref[...])


def _f32(shape):
    return jax.ShapeDtypeStruct(shape, jnp.float32)


@functools.lru_cache(maxsize=None)
def _build():
    mesh = plsc.VectorSubcoreMesh(core_axis_name="c", subcore_axis_name="s")

    deg_k = functools.partial(
        pl.kernel, _deg_body,
        out_type=_f32((NC, ACC, 16)),
        mesh=mesh,
        compiler_params=pltpu.CompilerParams(use_tc_tiling_on_sc=False),
        scratch_types=[
            pltpu.VMEM((S, EB), jnp.int32),
            pltpu.VMEM((EB, 16), jnp.float32),
            pltpu.VMEM((EB, 16), jnp.float32),
            pltpu.VMEM_SHARED((ACC, 16), jnp.float32),
        ],
    )()

    spmm_k = functools.partial(
        pl.kernel, _spmm_body,
        out_type=_f32((NC, ACC, DH)),
        mesh=mesh,
        compiler_params=pltpu.CompilerParams(use_tc_tiling_on_sc=False),
        scratch_types=[
            pltpu.VMEM((S2 + NB, EB2), jnp.int32),
            pltpu.VMEM((S2, EB2), jnp.int32),
            pltpu.VMEM((EB2, DH), jnp.float32),
            pltpu.VMEM((EB2, DH), jnp.float32),
            pltpu.VMEM_SHARED((ACC, DH), jnp.float32),
            pltpu.SemaphoreType.DMA,
            pltpu.SemaphoreType.DMA,
        ],
    )()

    pool_k = functools.partial(
        pl.kernel, _pool_body,
        out_type=(_f32((NC, 128, D)), _f32((NC, 128, D))),
        mesh=mesh,
        compiler_params=pltpu.CompilerParams(use_tc_tiling_on_sc=False),
        scratch_types=[
            pltpu.VMEM((PB, EB), jnp.int32),
            pltpu.VMEM((PB, EB), jnp.int32),
            pltpu.VMEM((EB, D), jnp.float32),
            pltpu.VMEM((EB, D), jnp.float32),
            pltpu.VMEM((8, D), jnp.float32),
            pltpu.VMEM_SHARED((128, D), jnp.float32),
            pltpu.VMEM_SHARED((128, D), jnp.float32),
            pltpu.SemaphoreType.DMA,
        ],
    )()

    grid = (N // BLK,)
    vec_spec = pl.BlockSpec((1, D), lambda j: (0, 0))
    row_spec = pl.BlockSpec((BLK, D), lambda j: (j, 0))
    mat_spec = pl.BlockSpec((D, D), lambda j: (0, 0))
    p_spec = pl.BlockSpec((NC, BLK, DH), lambda j: (0, j, 0))
    hw_spec = pl.BlockSpec((NC, BLK, DH), lambda j: (0, j, 0))
    hw_shape = _f32((NC, N, DH))

    first_k = pl.pallas_call(
        _first_body,
        grid=grid,
        in_specs=[row_spec,
                  pl.BlockSpec((NC, BLK, 16), lambda j: (0, j, 0)),
                  mat_spec, vec_spec],
        out_specs=[row_spec, hw_spec],
        out_shape=[_f32((N, D)), hw_shape],
    )

    mid_k = pl.pallas_call(
        _mid_body,
        grid=grid,
        in_specs=[p_spec, row_spec, row_spec, vec_spec, vec_spec,
                  mat_spec, vec_spec],
        out_specs=[row_spec, hw_spec],
        out_shape=[_f32((N, D)), hw_shape],
    )

    last_k = pl.pallas_call(
        _last_body,
        grid=grid,
        in_specs=[p_spec, row_spec, row_spec, vec_spec, vec_spec],
        out_specs=row_spec,
        out_shape=_f32((N, D)),
    )

    head_k = pl.pallas_call(
        _head_body,
        in_specs=[pl.BlockSpec((NC, 128, D), lambda: (0, 0, 0)),
                  pl.BlockSpec((NC, 128, D), lambda: (0, 0, 0)),
                  pl.BlockSpec((D, D), lambda: (0, 0)),
                  pl.BlockSpec((1, D), lambda: (0, 0))],
        out_specs=pl.BlockSpec((128, D), lambda: (0, 0)),
        out_shape=_f32((128, D)),
    )

    return deg_k, spmm_k, pool_k, first_k, mid_k, last_k, head_k


def kernel(x, edge_index, batch, W1, b1, g1, bt1, W2, b2, g2, bt2,
           W3, b3, g3, bt3, Wc, bc):
    deg_k, spmm_k, pool_k, first_k, mid_k, last_k, head_k = _build()

    loop = jnp.arange(N, dtype=jnp.int32)
    src = jnp.concatenate([edge_index[0], loop])
    dst = jnp.concatenate([edge_index[1], loop])
    dsts = jnp.pad(dst, (0, ECAP - ET),
                   constant_values=N).reshape(NW, S, EB)
    srcs2 = jnp.pad(src, (0, ECAP2 - ET)).reshape(NS, S2, EB2)
    srcs2 = jnp.concatenate(
        [srcs2, jnp.zeros((NS, NB, EB2), jnp.int32)], axis=1)
    dsts2 = jnp.pad(dst, (0, ECAP2 - ET),
                    constant_values=N).reshape(NS, S2, EB2)

    degp = deg_k(dsts)

    b1r = b1.reshape(1, D)
    dinvf, hw = first_k(x, degp, W1, b1r)

    p = spmm_k(srcs2, dsts2, hw)
    h1, hw = mid_k(p, x, dinvf, g1.reshape(1, D), bt1.reshape(1, D),
                   W2, b2.reshape(1, D))
    p = spmm_k(srcs2, dsts2, hw)
    h2, hw = mid_k(p, h1, dinvf, g2.reshape(1, D), bt2.reshape(1, D),
                   W3, b3.reshape(1, D))
    p = spmm_k(srcs2, dsts2, hw)
    h3 = last_k(p, h2, dinvf, g3.reshape(1, D), bt3.reshape(1, D))

    rows = jnp.minimum(jnp.arange(PCAP, dtype=jnp.int32), N - 1)
    rows = rows.reshape(NW, PB, EB)
    bpad = jnp.pad(batch, (0, PCAP - N), constant_values=G).reshape(NW, PB, EB)
    sums, cnts = pool_k(rows, bpad, h3)

    wcp = jnp.pad(Wc, ((0, 0), (0, D - NCLS)))
    bcp = jnp.pad(bc, (0, D - NCLS)).reshape(1, D)
    logits = head_k(sums, cnts, wcp, bcp)
    return logits[:G, :NCLS]


# speedup vs baseline: 1.6765x; 1.1028x over previous
import functools

import jax
import jax.numpy as jnp
from jax import lax
from jax.experimental import pallas as pl
from jax.experimental.pallas import tpu as pltpu
from jax.experimental.pallas import tpu_sc as plsc

N = 10000
E = 320000
D = 128
G = 64
NCLS = 10
NC, NS = 2, 16
NW = NC * NS
EB = 128
ET = E + N
S = -(-ET // (NW * EB))
S += S % 2
ECAP = NW * S * EB
DH = D // 2
EB2 = 128
NB = 2
S2 = -(-ET // (NS * EB2))
S2 += (-S2) % NB
ECAP2 = NS * S2 * EB2
ACC = 10240
RPT = ACC // NS
N2 = ACC
PR = N2 // NW
PC = 64
PB = PR // PC
BLK = 1024
C0 = float((1.0 + 1e-5) ** -0.5)


def _deg_body(dsts, degp, didx, onesb, zb, deg_sp):
    c = lax.axis_index("c")
    s = lax.axis_index("s")
    wid = c * NS + s

    def fill(i, carry):
        zb[i, pl.ds(0, 16)] = jnp.zeros((16,), jnp.float32)
        onesb[i, pl.ds(0, 16)] = jnp.ones((16,), jnp.float32)
        return carry

    lax.fori_loop(0, EB, fill, 0)
    for r in range(RPT // EB):
        pltpu.sync_copy(zb, deg_sp.at[pl.ds(s * RPT + r * EB, EB)])
    plsc.subcore_barrier()
    pltpu.sync_copy(dsts.at[wid], didx)

    def body(j, carry):
        pltpu.sync_copy(onesb, deg_sp.at[didx.at[j]], add=True)
        return carry

    lax.fori_loop(0, S, body, 0)
    plsc.subcore_barrier()
    for r in range(RPT // EB):
        pltpu.sync_copy(deg_sp.at[pl.ds(s * RPT + r * EB, EB)],
                        degp.at[c, pl.ds(s * RPT + r * EB, EB)])


def _spmm_body(srcs, dsts, hw, outp, sidx, didx,
               buf0, buf1, acc_sp, sem0, sem1):
    c = lax.axis_index("c")
    s = lax.axis_index("s")
    bufs = (buf0, buf1)
    sems = (sem0, sem1)

    def zfill(i, carry):
        for k in range(DH // 16):
            buf0[i, pl.ds(k * 16, 16)] = jnp.zeros((16,), jnp.float32)
        return carry

    lax.fori_loop(0, EB2, zfill, 0)
    for r in range(RPT // EB2):
        pltpu.sync_copy(buf0, acc_sp.at[pl.ds(s * RPT + r * EB2, EB2)])
    plsc.subcore_barrier()
    pltpu.sync_copy(srcs.at[s], sidx)
    pltpu.sync_copy(dsts.at[s], didx)

    for b in range(NB):
        pltpu.async_copy(hw.at[c].at[sidx.at[b]], bufs[b], sems[b])

    def body(i, carry):
        j = NB * i
        for b in range(NB):
            pltpu.make_async_copy(hw.at[c].at[sidx.at[j + b]],
                                  bufs[b], sems[b]).wait()
            pltpu.sync_copy(bufs[b], acc_sp.at[didx.at[j + b]], add=True)
            pltpu.async_copy(hw.at[c].at[sidx.at[j + b + NB]],
                             bufs[b], sems[b])
        return carry

    lax.fori_loop(0, S2 // NB, body, 0)
    for b in range(NB):
        pltpu.make_async_copy(hw.at[c].at[sidx.at[0]], bufs[b], sems[b]).wait()
    plsc.subcore_barrier()
    pltpu.sync_copy(acc_sp.at[pl.ds(s * RPT, RPT)],
                    outp.at[c, pl.ds(s * RPT, RPT)])


def _pool_body(bidx_h, h3, outs, bidx, hbuf, zb, sums_sp, sem):
    c = lax.axis_index("c")
    s = lax.axis_index("s")
    wid = c * NS + s

    def zfill(i, carry):
        for k in range(8):
            zb[i, pl.ds(k * 16, 16)] = jnp.zeros((16,), jnp.float32)
        return carry

    lax.fori_loop(0, 8, zfill, 0)
    pltpu.sync_copy(zb, sums_sp.at[pl.ds(s * 8, 8)])
    plsc.subcore_barrier()
    pltpu.sync_copy(bidx_h.at[wid], bidx)
    pltpu.async_copy(h3.at[pl.ds(wid * PR, PR)], hbuf, sem).wait()
    for k in range(PB):
        pltpu.sync_copy(hbuf.at[pl.ds(k * PC, PC)],
                        sums_sp.at[bidx.at[k]], add=True)
    plsc.subcore_barrier()

    @pl.when(s == 0)
    def _():
        pltpu.sync_copy(sums_sp, outs.at[c])


def _first_body(x_ref, degp_ref, w_ref, b_ref, dinv_ref, hw_ref):
    deg = degp_ref[0, :, 0:1] + degp_ref[1, :, 0:1]
    dinv = jnp.where(deg > 0, lax.rsqrt(jnp.maximum(deg, 1e-30)), 0.0)
    dinvb = jnp.broadcast_to(dinv, (BLK, D))
    dinv_ref[...] = dinvb
    hw = jnp.dot(x_ref[...], w_ref[...], preferred_element_type=jnp.float32)
    hw = (hw + b_ref[...]) * dinvb
    hw_ref[0] = hw[:, :DH]
    hw_ref[1] = hw[:, DH:]


def _mid_body(p_ref, prev_ref, dinv_ref, g_ref, bt_ref, w_ref, b_ref,
              hn_ref, hw_ref):
    dinv = dinv_ref[...]
    sv = jnp.concatenate([p_ref[0], p_ref[1]], axis=-1) * dinv
    hn = jnp.maximum(g_ref[...] * C0 * sv + bt_ref[...], 0.0) + prev_ref[...]
    hn_ref[...] = hn
    hw = jnp.dot(hn, w_ref[...], preferred_element_type=jnp.float32)
    hw = (hw + b_ref[...]) * dinv
    hw_ref[0] = hw[:, :DH]
    hw_ref[1] = hw[:, DH:]


def _last_body(p_ref, prev_ref, dinv_ref, g_ref, bt_ref, hn_ref):
    sv = jnp.concatenate([p_ref[0], p_ref[1]], axis=-1) * dinv_ref[...]
    hn_ref[...] = (jnp.maximum(g_ref[...] * C0 * sv + bt_ref[...], 0.0)
                   + prev_ref[...])


def _head_body(sums_ref, batch_ref, wc_ref, bc_ref, out_ref):
    sv = sums_ref[0] + sums_ref[1]
    gid = lax.broadcasted_iota(jnp.int32, (128, 1), 0)
    cv = jnp.sum((batch_ref[...] == gid).astype(jnp.float32), axis=1,
                 keepdims=True)
    emb = sv / jnp.maximum(cv, 1.0)
    out_ref[...] = (jnp.dot(emb, wc_ref[...], preferred_element_type=jnp.float32)
                    + bc_ref[...])


def _f32(shape):
    return jax.ShapeDtypeStruct(shape, jnp.float32)


@functools.lru_cache(maxsize=None)
def _build():
    mesh = plsc.VectorSubcoreMesh(core_axis_name="c", subcore_axis_name="s")

    deg_k = functools.partial(
        pl.kernel, _deg_body,
        out_type=_f32((NC, ACC, 16)),
        mesh=mesh,
        compiler_params=pltpu.CompilerParams(use_tc_tiling_on_sc=False),
        scratch_types=[
            pltpu.VMEM((S, EB), jnp.int32),
            pltpu.VMEM((EB, 16), jnp.float32),
            pltpu.VMEM((EB, 16), jnp.float32),
            pltpu.VMEM_SHARED((ACC, 16), jnp.float32),
        ],
    )()

    spmm_k = functools.partial(
        pl.kernel, _spmm_body,
        out_type=_f32((NC, ACC, DH)),
        mesh=mesh,
        compiler_params=pltpu.CompilerParams(use_tc_tiling_on_sc=False),
        scratch_types=[
            pltpu.VMEM((S2 + NB, EB2), jnp.int32),
            pltpu.VMEM((S2, EB2), jnp.int32),
            pltpu.VMEM((EB2, DH), jnp.float32),
            pltpu.VMEM((EB2, DH), jnp.float32),
            pltpu.VMEM_SHARED((ACC, DH), jnp.float32),
            pltpu.SemaphoreType.DMA,
            pltpu.SemaphoreType.DMA,
        ],
    )()

    pool_k = functools.partial(
        pl.kernel, _pool_body,
        out_type=_f32((NC, 128, D)),
        mesh=mesh,
        compiler_params=pltpu.CompilerParams(use_tc_tiling_on_sc=False),
        scratch_types=[
            pltpu.VMEM((PB, PC), jnp.int32),
            pltpu.VMEM((PR, D), jnp.float32),
            pltpu.VMEM((8, D), jnp.float32),
            pltpu.VMEM_SHARED((128, D), jnp.float32),
            pltpu.SemaphoreType.DMA,
        ],
    )()

    grid = (N2 // BLK,)
    vec_spec = pl.BlockSpec((1, D), lambda j: (0, 0))
    row_spec = pl.BlockSpec((BLK, D), lambda j: (j, 0))
    mat_spec = pl.BlockSpec((D, D), lambda j: (0, 0))
    p_spec = pl.BlockSpec((NC, BLK, DH), lambda j: (0, j, 0))
    hw_spec = pl.BlockSpec((NC, BLK, DH), lambda j: (0, j, 0))
    hw_shape = _f32((NC, N2, DH))

    first_k = pl.pallas_call(
        _first_body,
        grid=grid,
        in_specs=[row_spec,
                  pl.BlockSpec((NC, BLK, 16), lambda j: (0, j, 0)),
                  mat_spec, vec_spec],
        out_specs=[row_spec, hw_spec],
        out_shape=[_f32((N2, D)), hw_shape],
    )

    mid_k = pl.pallas_call(
        _mid_body,
        grid=grid,
        in_specs=[p_spec, row_spec, row_spec, vec_spec, vec_spec,
                  mat_spec, vec_spec],
        out_specs=[row_spec, hw_spec],
        out_shape=[_f32((N2, D)), hw_shape],
    )

    last_k = pl.pallas_call(
        _last_body,
        grid=grid,
        in_specs=[p_spec, row_spec, row_spec, vec_spec, vec_spec],
        out_specs=row_spec,
        out_shape=_f32((N2, D)),
    )

    head_k = pl.pallas_call(
        _head_body,
        in_specs=[pl.BlockSpec((NC, 128, D), lambda: (0, 0, 0)),
                  pl.BlockSpec((1, N2), lambda: (0, 0)),
                  pl.BlockSpec((D, D), lambda: (0, 0)),
                  pl.BlockSpec((1, D), lambda: (0, 0))],
        out_specs=pl.BlockSpec((128, D), lambda: (0, 0)),
        out_shape=_f32((128, D)),
    )

    return deg_k, spmm_k, pool_k, first_k, mid_k, last_k, head_k


def kernel(x, edge_index, batch, W1, b1, g1, bt1, W2, b2, g2, bt2,
           W3, b3, g3, bt3, Wc, bc):
    deg_k, spmm_k, pool_k, first_k, mid_k, last_k, head_k = _build()

    loop = jnp.arange(N, dtype=jnp.int32)
    src = jnp.concatenate([edge_index[0], loop])
    dst = jnp.concatenate([edge_index[1], loop])
    dsts = jnp.pad(dst, (0, ECAP - ET),
                   constant_values=N).reshape(NW, S, EB)
    srcs2 = jnp.pad(src, (0, ECAP2 - ET)).reshape(NS, S2, EB2)
    srcs2 = jnp.concatenate(
        [srcs2, jnp.zeros((NS, NB, EB2), jnp.int32)], axis=1)
    dsts2 = jnp.pad(dst, (0, ECAP2 - ET),
                    constant_values=N).reshape(NS, S2, EB2)

    degp = deg_k(dsts)

    b1r = b1.reshape(1, D)
    xp = jnp.pad(x, ((0, N2 - N), (0, 0)))
    dinvf, hw = first_k(xp, degp, W1, b1r)

    p = spmm_k(srcs2, dsts2, hw)
    h1, hw = mid_k(p, xp, dinvf, g1.reshape(1, D), bt1.reshape(1, D),
                   W2, b2.reshape(1, D))
    p = spmm_k(srcs2, dsts2, hw)
    h2, hw = mid_k(p, h1, dinvf, g2.reshape(1, D), bt2.reshape(1, D),
                   W3, b3.reshape(1, D))
    p = spmm_k(srcs2, dsts2, hw)
    h3 = last_k(p, h2, dinvf, g3.reshape(1, D), bt3.reshape(1, D))

    bpad = jnp.pad(batch, (0, N2 - N), constant_values=G)
    sums = pool_k(bpad.reshape(NW, PB, PC), h3)

    wcp = jnp.pad(Wc, ((0, 0), (0, D - NCLS)))
    bcp = jnp.pad(bc, (0, D - NCLS)).reshape(1, D)
    logits = head_k(sums, bpad.reshape(1, N2), wcp, bcp)
    return logits[:G, :NCLS]
